# Initial kernel scaffold; baseline (speedup 1.0000x reference)
import jax
import jax.numpy as jnp
from jax.experimental import pallas as pl


def kernel(positive_sample, context_tensor, emb_table, negative_sample_ids):
    raise NotImplementedError("write your pallas kernel here")


# speedup vs baseline: 5.2967x; 5.2967x over previous
import functools

import jax
import jax.numpy as jnp
from jax import lax
from jax.experimental import pallas as pl
from jax.experimental.pallas import tpu as pltpu
from jax.experimental.pallas import tpu_sc as plsc

S, B, D, K, V = 50, 1024, 128, 10, 100000
SB = S * B
N = SB * K
NC, NS = 2, 16
NW = NC * NS
PAIRS_PER_W = SB // NW
CP = 64
CK = CP * K
NCHUNK = PAIRS_PER_W // CP
G = 128
NSTREAM = CK // G
LANES = 16
DSUB = D // LANES


def _sc_neg_dots(ids_hbm, ctx_hbm, table_hbm, out_hbm, idx_v, ctx_v, rows_v,
                 dots_v, sem):
    wid = lax.axis_index("s") * NC + lax.axis_index("c")
    pair0 = wid * PAIRS_PER_W
    lane_iota = lax.iota(jnp.int32, LANES)
    lane_mask = [lane_iota == l for l in range(LANES)]

    def chunk_body(g, carry):
        pbase = pair0 + g * CP
        ibase = pl.multiple_of(pbase * K, 8)
        pltpu.sync_copy(ids_hbm.at[pl.ds(ibase, CK)], idx_v)
        pltpu.sync_copy(ctx_hbm.at[pl.ds(pbase, CP)], ctx_v)
        copies = [
            pltpu.async_copy(table_hbm.at[idx_v.at[pl.ds(j * G, G)]],
                             rows_v.at[pl.ds(j * G, G)], sem)
            for j in range(NSTREAM)
        ]
        for cp in copies:
            cp.wait()

        def group_body(q, carry2):
            rbase = q * (8 * K)
            res = [jnp.zeros((LANES,), jnp.float32) for _ in range(5)]
            for pp in range(8):
                p = q * 8 + pp
                c = [ctx_v[p, pl.ds(j * LANES, LANES)] for j in range(DSUB)]
                for k in range(K):
                    t = pp * K + k
                    acc = rows_v[rbase + t, pl.ds(0, LANES)] * c[0]
                    for j in range(1, DSUB):
                        acc = acc + rows_v[rbase + t,
                                           pl.ds(j * LANES, LANES)] * c[j]
                    v, l = divmod(t, LANES)
                    res[v] = jnp.where(lane_mask[l], jnp.sum(acc), res[v])
            for v in range(5):
                dots_v[pl.ds(rbase + v * LANES, LANES)] = res[v]
            return carry2

        lax.fori_loop(0, CP // 8, group_body, 0)
        pltpu.sync_copy(dots_v, out_hbm.at[pl.ds(ibase, CK)])
        return carry

    lax.fori_loop(0, NCHUNK, chunk_body, 0)


@functools.cache
def _make_neg_dots():
    return functools.partial(
        pl.kernel,
        mesh=plsc.VectorSubcoreMesh(core_axis_name="c", subcore_axis_name="s"),
        out_type=jax.ShapeDtypeStruct((N,), jnp.float32),
        compiler_params=pltpu.CompilerParams(needs_layout_passes=False),
        scratch_types=[
            pltpu.VMEM((CK,), jnp.int32),
            pltpu.VMEM((CP, D), jnp.float32),
            pltpu.VMEM((CK, D), jnp.float32),
            pltpu.VMEM((CK,), jnp.float32),
            pltpu.SemaphoreType.DMA,
        ],
    )(_sc_neg_dots)


def _combine_body(pos_ref, ctx_ref, dots_ref, out_ref):
    i = pl.program_id(0)

    @pl.when(i == 0)
    def _init():
        out_ref[0, 0] = 0.0

    pos_logits = jnp.sum(pos_ref[0] * ctx_ref[0], axis=-1)
    total = (jnp.sum(jax.nn.log_sigmoid(pos_logits))
             + jnp.sum(jax.nn.log_sigmoid(-dots_ref[0])))
    out_ref[0, 0] = out_ref[0, 0] - total


def kernel(positive_sample, context_tensor, emb_table, negative_sample_ids):
    ids32 = negative_sample_ids.astype(jnp.int32).reshape(N)
    ctx2d = context_tensor.reshape(SB, D)
    dots = _make_neg_dots()(ids32, ctx2d, emb_table)

    loss = pl.pallas_call(
        _combine_body,
        grid=(S,),
        in_specs=[
            pl.BlockSpec((1, B, D), lambda i: (i, 0, 0)),
            pl.BlockSpec((1, B, D), lambda i: (i, 0, 0)),
            pl.BlockSpec((1, B, K), lambda i: (i, 0, 0)),
        ],
        out_specs=pl.BlockSpec((1, 1), lambda i: (0, 0),
                               memory_space=pltpu.SMEM),
        out_shape=jax.ShapeDtypeStruct((1, 1), jnp.float32),
    )(positive_sample, context_tensor, dots.reshape(S, B, K))
    return loss[0, 0]
